# baseline (speedup 1.0000x reference)
import functools

import jax
import jax.numpy as jnp
import numpy as np
from jax import lax
from jax.experimental import pallas as pl
from jax.experimental.pallas import tpu as pltpu
from jax.experimental.pallas import tpu_sc as plsc

_D = 64
_ROW = 2 * _D
_EG = 0.57721566490153286
_EPS = 1e-23
_NC, _NS = 2, 16
_NW = _NC * _NS
_GCHUNK = 128
_L = 16


def _sc_gather(idx2d, table):
    batch = idx2d.shape[0] * 64
    e_per_w = batch // _NW
    n_chunks = e_per_w // _GCHUNK
    idx_rows_per_w = 2 * e_per_w // 128
    mesh = plsc.VectorSubcoreMesh(core_axis_name="c", subcore_axis_name="s")

    @functools.partial(
        pl.kernel,
        out_type=jax.ShapeDtypeStruct((2 * batch, _ROW), jnp.float32),
        mesh=mesh,
        scratch_types=[
            pltpu.VMEM((idx_rows_per_w, 128), jnp.int32),
            pltpu.VMEM((_GCHUNK, _ROW), jnp.float32),
            pltpu.VMEM((_GCHUNK, _ROW), jnp.float32),
            pltpu.SemaphoreType.DMA,
            pltpu.SemaphoreType.DMA,
        ],
    )
    def gather_kernel(idx_hbm, table_hbm, out_hbm, pairs_v,
                      rows_a, rows_b, sem_a, sem_b):
        wid = lax.axis_index("s") * _NC + lax.axis_index("c")
        base = wid * e_per_w
        pltpu.sync_copy(
            idx_hbm.at[pl.ds(wid * idx_rows_per_w, idx_rows_per_w), :],
            pairs_v)
        jobs = []
        for j in range(n_chunks):
            jobs.append((pairs_v.at[2 * j], base + j * _GCHUNK))
        for j in range(n_chunks):
            jobs.append((pairs_v.at[2 * j + 1], batch + base + j * _GCHUNK))
        bufs = ((rows_a, sem_a), (rows_b, sem_b))
        pltpu.async_copy(table_hbm.at[jobs[0][0]], rows_a, sem_a)
        for j, (idx_ref, dst_off) in enumerate(jobs):
            buf, sem = bufs[j % 2]
            nbuf, nsem = bufs[(j + 1) % 2]
            if j + 1 < len(jobs):
                pltpu.async_copy(table_hbm.at[jobs[j + 1][0]], nbuf, nsem)
            pltpu.make_async_copy(table_hbm.at[idx_ref], buf, sem).wait()
            pltpu.sync_copy(buf, out_hbm.at[pl.ds(dst_off, _GCHUNK)])

    return gather_kernel(idx2d, table)


def _tc_compute(staging, batch):
    blk = 1024
    grid = batch // blk
    inv_k = float(np.exp(-2.0 * _EG))

    def body(sub_ref, sup_ref, o_ref):
        sub = sub_ref[...]
        sup = sup_ref[...]
        ea = jnp.exp(sub[:, :_D])
        eb = jnp.exp(sup[:, :_D])
        pda = 1.0 + jnp.exp(sub[:, _D:])
        pdb = 1.0 + jnp.exp(sup[:, _D:])
        big_a = ea * pda
        big_b = eb * pdb
        s = ea + eb
        t = big_a + big_b
        pm = big_a * big_b
        num = jnp.log1p(pm / (t * s) * inv_k) + _EPS
        den = jnp.log1p(pda * inv_k) + _EPS
        lsum = jnp.sum(jnp.log(num / den), axis=-1)
        o_ref[...] = jnp.clip(jnp.exp(lsum), 0.0, 1.0)

    return pl.pallas_call(
        body,
        grid=(grid,),
        in_specs=[
            pl.BlockSpec((blk, _ROW), lambda i: (i, 0)),
            pl.BlockSpec((blk, _ROW), lambda i: (i + grid, 0)),
        ],
        out_specs=pl.BlockSpec((blk,), lambda i: (i,)),
        out_shape=jax.ShapeDtypeStruct((batch,), jnp.float32),
    )(staging, staging)


def kernel(gci, table):
    batch = gci.shape[0]
    idx2d = (gci.T.reshape(2, batch // 128, 128)
             .transpose(1, 0, 2).reshape(batch // 64, 128))
    staging = _sc_gather(idx2d, table)
    return _tc_compute(staging, batch)


# speedup vs baseline: 1.0057x; 1.0057x over previous
import functools

import jax
import jax.numpy as jnp
import numpy as np
from jax import lax
from jax.experimental import pallas as pl
from jax.experimental.pallas import tpu as pltpu
from jax.experimental.pallas import tpu_sc as plsc

_D = 64
_ROW = 2 * _D
_EG = 0.57721566490153286
_EPS = 1e-23
_NC, _NS = 2, 16
_NW = _NC * _NS
_GCHUNK = 128
_L = 16


def _sc_gather(idx2d, table):
    batch = idx2d.shape[0] * 64
    e_per_w = batch // _NW
    n_chunks = e_per_w // _GCHUNK
    idx_rows_per_w = 2 * e_per_w // 128
    mesh = plsc.VectorSubcoreMesh(core_axis_name="c", subcore_axis_name="s")

    @functools.partial(
        pl.kernel,
        out_type=jax.ShapeDtypeStruct((2 * batch, _ROW), jnp.float32),
        mesh=mesh,
        scratch_types=[
            pltpu.VMEM((idx_rows_per_w, 128), jnp.int32),
            pltpu.VMEM((_GCHUNK, _ROW), jnp.float32),
            pltpu.VMEM((_GCHUNK, _ROW), jnp.float32),
            pltpu.SemaphoreType.DMA,
            pltpu.SemaphoreType.DMA,
        ],
    )
    def gather_kernel(idx_hbm, table_hbm, out_hbm, pairs_v,
                      rows_a, rows_b, sem_a, sem_b):
        wid = lax.axis_index("s") * _NC + lax.axis_index("c")
        base = wid * e_per_w
        pltpu.sync_copy(
            idx_hbm.at[pl.ds(wid * idx_rows_per_w, idx_rows_per_w), :],
            pairs_v)
        jobs = []
        for j in range(n_chunks):
            jobs.append((pairs_v.at[2 * j], base + j * _GCHUNK))
        for j in range(n_chunks):
            jobs.append((pairs_v.at[2 * j + 1], batch + base + j * _GCHUNK))
        bufs = ((rows_a, sem_a), (rows_b, sem_b))
        pltpu.async_copy(table_hbm.at[jobs[0][0]], rows_a, sem_a)
        for j, (idx_ref, dst_off) in enumerate(jobs):
            buf, sem = bufs[j % 2]
            nbuf, nsem = bufs[(j + 1) % 2]
            if j + 1 < len(jobs):
                pltpu.async_copy(table_hbm.at[jobs[j + 1][0]], nbuf, nsem)
            pltpu.make_async_copy(table_hbm.at[idx_ref], buf, sem).wait()
            pltpu.sync_copy(buf, out_hbm.at[pl.ds(dst_off, _GCHUNK)])

    return gather_kernel(idx2d, table)


def _tc_compute(staging, batch):
    blk = 4096
    grid = batch // blk
    inv_k = float(np.exp(-2.0 * _EG))

    def body(sub_ref, sup_ref, o_ref):
        sub = sub_ref[...]
        sup = sup_ref[...]
        ea = jnp.exp(sub[:, :_D])
        eb = jnp.exp(sup[:, :_D])
        pda = 1.0 + jnp.exp(sub[:, _D:])
        pdb = 1.0 + jnp.exp(sup[:, _D:])
        big_a = ea * pda
        big_b = eb * pdb
        s = ea + eb
        t = big_a + big_b
        pm = big_a * big_b
        num = jnp.log1p(pm / (t * s) * inv_k) + _EPS
        den = jnp.log1p(pda * inv_k) + _EPS
        lsum = jnp.sum(jnp.log(num / den), axis=-1)
        o_ref[...] = jnp.clip(jnp.exp(lsum), 0.0, 1.0)

    return pl.pallas_call(
        body,
        grid=(grid,),
        in_specs=[
            pl.BlockSpec((blk, _ROW), lambda i: (i, 0)),
            pl.BlockSpec((blk, _ROW), lambda i: (i + grid, 0)),
        ],
        out_specs=pl.BlockSpec((blk,), lambda i: (i,)),
        out_shape=jax.ShapeDtypeStruct((batch,), jnp.float32),
    )(staging, staging)


def kernel(gci, table):
    batch = gci.shape[0]
    idx2d = (gci.T.reshape(2, batch // 128, 128)
             .transpose(1, 0, 2).reshape(batch // 64, 128))
    staging = _sc_gather(idx2d, table)
    return _tc_compute(staging, batch)


# speedup vs baseline: 1.0089x; 1.0032x over previous
import functools

import jax
import jax.numpy as jnp
import numpy as np
from jax import lax
from jax.experimental import pallas as pl
from jax.experimental.pallas import tpu as pltpu
from jax.experimental.pallas import tpu_sc as plsc

_D = 64
_ROW = 2 * _D
_EG = 0.57721566490153286
_EPS = 1e-23
_NC, _NS = 2, 16
_NW = _NC * _NS
_GCHUNK = 128
_L = 16


def _sc_gather(idx2d, table):
    batch = idx2d.shape[0] * 64
    e_per_w = batch // _NW
    n_chunks = e_per_w // _GCHUNK
    idx_rows_per_w = 2 * e_per_w // 128
    mesh = plsc.VectorSubcoreMesh(core_axis_name="c", subcore_axis_name="s")

    @functools.partial(
        pl.kernel,
        out_type=jax.ShapeDtypeStruct((2 * batch, _ROW), jnp.float32),
        mesh=mesh,
        scratch_types=[
            pltpu.VMEM((idx_rows_per_w, 128), jnp.int32),
            pltpu.VMEM((_GCHUNK, _ROW), jnp.float32),
            pltpu.VMEM((_GCHUNK, _ROW), jnp.float32),
            pltpu.SemaphoreType.DMA,
            pltpu.SemaphoreType.DMA,
        ],
    )
    def gather_kernel(idx_hbm, table_hbm, out_hbm, pairs_v,
                      rows_a, rows_b, sem_a, sem_b):
        wid = lax.axis_index("s") * _NC + lax.axis_index("c")
        base = wid * e_per_w
        pltpu.sync_copy(
            idx_hbm.at[pl.ds(wid * idx_rows_per_w, idx_rows_per_w), :],
            pairs_v)
        jobs = []
        for j in range(n_chunks):
            jobs.append((pairs_v.at[2 * j], base + j * _GCHUNK))
        for j in range(n_chunks):
            jobs.append((pairs_v.at[2 * j + 1], batch + base + j * _GCHUNK))
        bufs = ((rows_a, sem_a), (rows_b, sem_b))
        pltpu.async_copy(table_hbm.at[jobs[0][0]], rows_a, sem_a)
        for j, (idx_ref, dst_off) in enumerate(jobs):
            buf, sem = bufs[j % 2]
            nbuf, nsem = bufs[(j + 1) % 2]
            if j + 1 < len(jobs):
                pltpu.async_copy(table_hbm.at[jobs[j + 1][0]], nbuf, nsem)
            pltpu.make_async_copy(table_hbm.at[idx_ref], buf, sem).wait()
            pltpu.sync_copy(buf, out_hbm.at[pl.ds(dst_off, _GCHUNK)])

    return gather_kernel(idx2d, table)


def _tc_compute(staging, batch):
    blk = 4096
    grid = batch // blk
    inv_k = float(np.exp(-2.0 * _EG))

    def body(sub_ref, sup_ref, o_ref):
        sub = sub_ref[...]
        sup = sup_ref[...]
        q = jnp.exp(sub[:, :_D] - sup[:, :_D])
        pda = 1.0 + jnp.exp(sub[:, _D:])
        pdb = 1.0 + jnp.exp(sup[:, _D:])
        qa = q * pda
        m = qa * pdb / ((qa + pdb) * (1.0 + q))
        num = jnp.log1p(m * inv_k) + _EPS
        den = jnp.log1p(pda * inv_k) + _EPS
        lsum = jnp.sum(jnp.log(num / den), axis=-1)
        o_ref[...] = jnp.clip(jnp.exp(lsum), 0.0, 1.0)

    return pl.pallas_call(
        body,
        grid=(grid,),
        in_specs=[
            pl.BlockSpec((blk, _ROW), lambda i: (i, 0)),
            pl.BlockSpec((blk, _ROW), lambda i: (i + grid, 0)),
        ],
        out_specs=pl.BlockSpec((blk,), lambda i: (i,)),
        out_shape=jax.ShapeDtypeStruct((batch,), jnp.float32),
    )(staging, staging)


def kernel(gci, table):
    batch = gci.shape[0]
    idx2d = (gci.T.reshape(2, batch // 128, 128)
             .transpose(1, 0, 2).reshape(batch // 64, 128))
    staging = _sc_gather(idx2d, table)
    return _tc_compute(staging, batch)


# speedup vs baseline: 1.0373x; 1.0282x over previous
import functools

import jax
import jax.numpy as jnp
import numpy as np
from jax import lax
from jax.experimental import pallas as pl
from jax.experimental.pallas import tpu as pltpu
from jax.experimental.pallas import tpu_sc as plsc

_D = 64
_ROW = 2 * _D
_EG = 0.57721566490153286
_EPS = 1e-23
_NC, _NS = 2, 16
_NW = _NC * _NS
_GCHUNK = 128
_L = 16


def _sc_gather(idx2d, table, half, nhalves):
    batch = idx2d.shape[0] * 64 // nhalves
    e_per_w = batch // _NW
    n_chunks = e_per_w // _GCHUNK
    idx_rows_per_w = 2 * e_per_w // 128
    idx_row0 = half * 2 * batch // 128
    mesh = plsc.VectorSubcoreMesh(core_axis_name="c", subcore_axis_name="s")

    @functools.partial(
        pl.kernel,
        out_type=jax.ShapeDtypeStruct((2 * batch, _ROW), jnp.float32),
        mesh=mesh,
        scratch_types=[
            pltpu.VMEM((idx_rows_per_w, 128), jnp.int32),
            pltpu.VMEM((_GCHUNK, _ROW), jnp.float32),
            pltpu.VMEM((_GCHUNK, _ROW), jnp.float32),
            pltpu.SemaphoreType.DMA,
            pltpu.SemaphoreType.DMA,
        ],
    )
    def gather_kernel(idx_hbm, table_hbm, out_hbm, pairs_v,
                      rows_a, rows_b, sem_a, sem_b):
        wid = lax.axis_index("s") * _NC + lax.axis_index("c")
        base = wid * e_per_w
        pltpu.sync_copy(
            idx_hbm.at[pl.ds(idx_row0 + wid * idx_rows_per_w,
                             idx_rows_per_w), :],
            pairs_v)
        jobs = []
        for j in range(n_chunks):
            jobs.append((pairs_v.at[2 * j], base + j * _GCHUNK))
        for j in range(n_chunks):
            jobs.append((pairs_v.at[2 * j + 1], batch + base + j * _GCHUNK))
        bufs = ((rows_a, sem_a), (rows_b, sem_b))
        pltpu.async_copy(table_hbm.at[jobs[0][0]], rows_a, sem_a)
        for j, (idx_ref, dst_off) in enumerate(jobs):
            buf, sem = bufs[j % 2]
            nbuf, nsem = bufs[(j + 1) % 2]
            if j + 1 < len(jobs):
                pltpu.async_copy(table_hbm.at[jobs[j + 1][0]], nbuf, nsem)
            pltpu.make_async_copy(table_hbm.at[idx_ref], buf, sem).wait()
            pltpu.sync_copy(buf, out_hbm.at[pl.ds(dst_off, _GCHUNK)])

    return gather_kernel(idx2d, table)


def _tc_compute(staging, batch):
    blk = 4096
    grid = batch // blk
    inv_k = float(np.exp(-2.0 * _EG))

    def body(sub_ref, sup_ref, o_ref):
        sub = sub_ref[...]
        sup = sup_ref[...]
        q = jnp.exp(sub[:, :_D] - sup[:, :_D])
        pda = 1.0 + jnp.exp(sub[:, _D:])
        pdb = 1.0 + jnp.exp(sup[:, _D:])
        qa = q * pda
        m = qa * pdb / ((qa + pdb) * (1.0 + q))
        num = jnp.log1p(m * inv_k) + _EPS
        den = jnp.log1p(pda * inv_k) + _EPS
        lsum = jnp.sum(jnp.log(num / den), axis=-1)
        o_ref[...] = jnp.clip(jnp.exp(lsum), 0.0, 1.0)

    return pl.pallas_call(
        body,
        grid=(grid,),
        in_specs=[
            pl.BlockSpec((blk, _ROW), lambda i: (i, 0)),
            pl.BlockSpec((blk, _ROW), lambda i: (i + grid, 0)),
        ],
        out_specs=pl.BlockSpec((blk,), lambda i: (i,)),
        out_shape=jax.ShapeDtypeStruct((batch,), jnp.float32),
    )(staging, staging)


def kernel(gci, table):
    batch = gci.shape[0]
    idx2d = (gci.T.reshape(2, batch // 128, 128)
             .transpose(1, 0, 2).reshape(batch // 64, 128))
    nh = 2
    stagings = [_sc_gather(idx2d, table, h, nh) for h in range(nh)]
    outs = [_tc_compute(s, batch // nh) for s in stagings]
    return jnp.concatenate(outs)
